# baseline (speedup 1.0000x reference)
import functools

import jax
import jax.numpy as jnp
from jax import lax
from jax.experimental import pallas as pl
from jax.experimental.pallas import tpu as pltpu
from jax.experimental.pallas import tpu_sc as plsc

VOCAB = 1000000
D = 64
B = 16384
K = 20
NW = 32
BPW = B // NW
NB = 32
NCH = BPW // NB
NROWS = NB * K
NOUT = B * (K + 1)


def _sc_body(u_hbm, v_hbm, pu_hbm, pv_hbm, nvt_hbm, out_hbm,
             uidx, vidx, nidx, ub0, vb0, nb0, ub1, vb1, nb1,
             outp, outn, sem0, sem1):
    wid = lax.axis_index("s") * 2 + lax.axis_index("c")
    iota = lax.iota(jnp.int32, 16)

    pltpu.sync_copy(pu_hbm.at[pl.ds(wid * BPW, BPW)], uidx)
    pltpu.sync_copy(pv_hbm.at[pl.ds(wid * BPW, BPW)], vidx)
    for k in range(K):
        pltpu.sync_copy(nvt_hbm.at[k, pl.ds(wid * BPW, BPW)], nidx.at[k])

    bufs = ((ub0, vb0, nb0, sem0), (ub1, vb1, nb1, sem1))

    def dmas(c, par):
        ub, vb, nb, sem = bufs[par]
        yield (u_hbm.at[uidx.at[pl.ds(c * NB, NB)]], ub, sem)
        yield (v_hbm.at[vidx.at[pl.ds(c * NB, NB)]], vb, sem)
        for k in range(K):
            yield (v_hbm.at[nidx.at[k, pl.ds(c * NB, NB)]],
                   nb.at[pl.ds(k * NB, NB)], sem)

    def fire(c, par):
        for s, d, m in dmas(c, par):
            pltpu.async_copy(s, d, m)

    def wait(c, par):
        for s, d, m in dmas(c, par):
            pltpu.make_async_copy(s, d, m).wait()

    xs = tuple(jnp.bitwise_xor(iota, 1 << t) for t in range(4))
    m0 = iota == 0

    def redsum(acc):
        for x in xs:
            acc = acc + acc.at[x].get(mode="promise_in_bounds")
        return acc

    def compute(c, par):
        ub, vb, nb, _ = bufs[par]

        @plsc.parallel_loop(0, NB, unroll=2)
        def _(bb):
            u = [ub[bb, pl.ds(16 * j, 16)] for j in range(4)]
            v = [vb[bb, pl.ds(16 * j, 16)] for j in range(4)]
            r = redsum(u[0] * v[0] + u[1] * v[1] + u[2] * v[2] + u[3] * v[3])
            plsc.store_scatter(
                outp, [jnp.full((16,), c * NB + bb, jnp.int32)], r, mask=m0)
            for k in range(K):
                row = k * NB + bb
                n = [nb[row, pl.ds(16 * j, 16)] for j in range(4)]
                r = redsum(u[0] * n[0] + u[1] * n[1] + u[2] * n[2] + u[3] * n[3])
                plsc.store_scatter(
                    outn, [jnp.full((16,), c * NROWS + row, jnp.int32)],
                    -r, mask=m0)

    fire(0, 0)

    def pair_body(c2, _):
        c = c2 * 2
        fire(c + 1, 1)
        wait(c, 0)
        compute(c, 0)

        @pl.when(c + 2 < NCH)
        def _():
            fire(c + 2, 0)
        wait(c + 1, 1)
        compute(c + 1, 1)
        return 0

    lax.fori_loop(0, NCH // 2, pair_body, 0)

    pltpu.sync_copy(outp, out_hbm.at[pl.ds(wid * BPW, BPW)])
    pltpu.sync_copy(outn, out_hbm.at[pl.ds(B + wid * BPW * K, BPW * K)])


_sc_dots = functools.partial(
    pl.kernel,
    out_type=jax.ShapeDtypeStruct((NOUT,), jnp.float32),
    mesh=plsc.VectorSubcoreMesh(core_axis_name="c", subcore_axis_name="s"),
    compiler_params=pltpu.CompilerParams(
        needs_layout_passes=False, use_tc_tiling_on_sc=False),
    scratch_types=[
        pltpu.VMEM((BPW,), jnp.int32),
        pltpu.VMEM((BPW,), jnp.int32),
        pltpu.VMEM((K, BPW), jnp.int32),
        pltpu.VMEM((NB, D), jnp.float32),
        pltpu.VMEM((NB, D), jnp.float32),
        pltpu.VMEM((NROWS, D), jnp.float32),
        pltpu.VMEM((NB, D), jnp.float32),
        pltpu.VMEM((NB, D), jnp.float32),
        pltpu.VMEM((NROWS, D), jnp.float32),
        pltpu.VMEM((BPW,), jnp.float32),
        pltpu.VMEM((BPW * K,), jnp.float32),
        pltpu.SemaphoreType.DMA,
        pltpu.SemaphoreType.DMA,
    ],
)(_sc_body)


def _tc_body(x_ref, o_ref):
    o_ref[0, 0] = -jnp.sum(jax.nn.log_sigmoid(x_ref[...]))


_tc_reduce = pl.pallas_call(
    _tc_body,
    out_shape=jax.ShapeDtypeStruct((1, 1), jnp.float32),
    out_specs=pl.BlockSpec(memory_space=pltpu.SMEM),
)


def kernel(u_weight, v_weight, pos_u, pos_v, neg_v):
    dots = _sc_dots(u_weight, v_weight,
                    pos_u.astype(jnp.int32), pos_v.astype(jnp.int32),
                    neg_v.astype(jnp.int32).T)
    loss = _tc_reduce(dots.reshape(NOUT // 1024, 1024))
    return loss[0, 0]
